# baseline (speedup 1.0000x reference)
import jax
import jax.numpy as jnp
from jax import lax
from jax.experimental import pallas as pl
from jax.experimental.pallas import tpu as pltpu
from jax.experimental.pallas import tpu_sc as plsc

N_NODES = 32
NC = 2
NS = 16
NW = NC * NS
L = 16
PAD = 16


def _sc_body(ev_hbm, logits_hbm, out_hbm, ev_v, tbl_v, out_v):
    wid = lax.axis_index("s") * NC + lax.axis_index("c")
    rows = ev_hbm.shape[0] // (N_NODES * NW)
    chunk = rows * N_NODES
    base = wid * chunk

    pltpu.sync_copy(ev_hbm.at[pl.ds(base, chunk)], ev_v.at[pl.ds(PAD, chunk)])
    pltpu.sync_copy(logits_hbm, tbl_v)

    one = jnp.ones((L,), jnp.float32)
    for j in range(N_NODES * 8 // L):
        x = tbl_v[pl.ds(j * L, L)]
        tbl_v[pl.ds(j * L, L)] = one / (one + jnp.exp(-x))

    lane = lax.iota(jnp.int32, L)
    zero = jnp.zeros((L,), jnp.int32)
    idx0 = lane * 8
    idx1 = idx0 + 8 * L
    m3 = lane >= 3
    m2 = lane >= 2
    m1 = lane >= 1

    @plsc.parallel_loop(0, rows, unroll=8)
    def row_body(r):
        p = PAD + r * N_NODES
        a = ev_v[pl.ds(p - 3, L)]
        b = ev_v[pl.ds(p - 2, L)]
        c = ev_v[pl.ds(p - 1, L)]
        conf0 = (jnp.where(m3, a << 2, zero)
                 + jnp.where(m2, b << 1, zero)
                 + jnp.where(m1, c, zero))
        out_v[pl.ds(r * N_NODES, L)] = plsc.load_gather(tbl_v, [conf0 + idx0])
        a1 = ev_v[pl.ds(p + 13, L)]
        b1 = ev_v[pl.ds(p + 14, L)]
        c1 = ev_v[pl.ds(p + 15, L)]
        conf1 = (a1 << 2) + (b1 << 1) + c1
        out_v[pl.ds(r * N_NODES + L, L)] = plsc.load_gather(tbl_v, [conf1 + idx1])

    pltpu.sync_copy(out_v, out_hbm.at[pl.ds(base, chunk)])


def kernel(evidence_tensor, logits):
    B, n = evidence_tensor.shape
    ev_flat = evidence_tensor.astype(jnp.int32).reshape(-1)
    logits_flat = logits.reshape(-1)
    chunk = B * n // NW

    mesh = plsc.VectorSubcoreMesh(core_axis_name="c", subcore_axis_name="s")
    out_flat = pl.kernel(
        _sc_body,
        out_type=jax.ShapeDtypeStruct((B * n,), jnp.float32),
        mesh=mesh,
        compiler_params=pltpu.CompilerParams(
            needs_layout_passes=False,
            disable_bounds_checks=True,
            disable_semaphore_checks=True,
            skip_device_barrier=True,
        ),
        scratch_types=[
            pltpu.VMEM((PAD + chunk,), jnp.int32),
            pltpu.VMEM((n * 8,), jnp.float32),
            pltpu.VMEM((chunk,), jnp.float32),
        ],
    )(ev_flat, logits_flat)
    return out_flat.reshape(B, n)


# speedup vs baseline: 1.1291x; 1.1291x over previous
import jax
import jax.numpy as jnp
from jax import lax
from jax.experimental import pallas as pl
from jax.experimental.pallas import tpu as pltpu
from jax.experimental.pallas import tpu_sc as plsc

N_NODES = 32
NC = 2
NS = 16
NW = NC * NS
L = 16


def _sc_body(ev_hbm, logits_hbm, out_hbm, ev_v, tbl_v, out_v):
    wid = lax.axis_index("s") * NC + lax.axis_index("c")
    rows = ev_hbm.shape[0] // NW
    half = rows // 2

    pltpu.sync_copy(logits_hbm, tbl_v)

    lane = lax.iota(jnp.int32, L)
    zero = jnp.zeros((L,), jnp.int32)
    one = jnp.ones((L,), jnp.float32)

    tcol = lane & 7
    for j in range(N_NODES * 8 // L):
        trow = (lane >> 3) + 2 * j
        x = plsc.load_gather(tbl_v, [trow, tcol])
        plsc.store_scatter(tbl_v, [trow, tcol], one / (one + jnp.exp(-x)))

    m3 = lane >= 3
    m2 = lane >= 2
    m1 = lane >= 1
    ca0 = jnp.maximum(lane - 3, zero)
    cb0 = jnp.maximum(lane - 2, zero)
    cc0 = jnp.maximum(lane - 1, zero)
    ca1 = lane + 13
    cb1 = lane + 14
    cc1 = lane + 15
    node1 = lane + L
    n = ev_hbm.shape[1]

    for h in range(2):
        base = wid * rows + h * half
        pltpu.sync_copy(ev_hbm.at[pl.ds(base, half), :],
                        ev_v.at[:, pl.ds(0, n)])

        @plsc.parallel_loop(0, half, unroll=8)
        def row_body(r):
            rr = jnp.full((L,), r, jnp.int32)
            a = plsc.load_gather(ev_v, [rr, ca0])
            b = plsc.load_gather(ev_v, [rr, cb0])
            c = plsc.load_gather(ev_v, [rr, cc0])
            conf0 = (jnp.where(m3, a << 2, zero)
                     + jnp.where(m2, b << 1, zero)
                     + jnp.where(m1, c, zero))
            out_v[r, pl.ds(0, L)] = plsc.load_gather(tbl_v, [lane, conf0])
            a1 = plsc.load_gather(ev_v, [rr, ca1])
            b1 = plsc.load_gather(ev_v, [rr, cb1])
            c1 = plsc.load_gather(ev_v, [rr, cc1])
            conf1 = (a1 << 2) + (b1 << 1) + c1
            out_v[r, pl.ds(L, L)] = plsc.load_gather(tbl_v, [node1, conf1])

        pltpu.sync_copy(out_v.at[:, pl.ds(0, n)],
                        out_hbm.at[pl.ds(base, half), :])


def kernel(evidence_tensor, logits):
    B, n = evidence_tensor.shape
    ev = evidence_tensor.astype(jnp.int32)
    rows = B // NW

    mesh = plsc.VectorSubcoreMesh(core_axis_name="c", subcore_axis_name="s")
    return pl.kernel(
        _sc_body,
        out_type=jax.ShapeDtypeStruct((B, n), jnp.float32),
        mesh=mesh,
        compiler_params=pltpu.CompilerParams(needs_layout_passes=False),
        scratch_types=[
            pltpu.VMEM((rows // 2, n), jnp.int32),
            pltpu.VMEM((n, 8), jnp.float32),
            pltpu.VMEM((rows // 2, n), jnp.float32),
        ],
    )(ev, logits)


# speedup vs baseline: 1.1321x; 1.0027x over previous
import jax
import jax.numpy as jnp
from jax import lax
from jax.experimental import pallas as pl
from jax.experimental.pallas import tpu as pltpu
from jax.experimental.pallas import tpu_sc as plsc

N_NODES = 32
NC = 2
NS = 16
NW = NC * NS
L = 16


def _sc_body(ev_hbm, logits_hbm, out_hbm, ev_v, tbl_v, out_v):
    wid = lax.axis_index("s") * NC + lax.axis_index("c")
    rows = ev_hbm.shape[0] // NW
    half = rows // 2

    pltpu.sync_copy(logits_hbm, tbl_v)

    lane = lax.iota(jnp.int32, L)
    zero = jnp.zeros((L,), jnp.int32)
    one = jnp.ones((L,), jnp.float32)

    tcol = lane & 7
    for j in range(N_NODES * 8 // L):
        trow = (lane >> 3) + 2 * j
        x = plsc.load_gather(tbl_v, [trow, tcol])
        plsc.store_scatter(tbl_v, [trow, tcol], one / (one + jnp.exp(-x)))

    m3 = lane >= 3
    m2 = lane >= 2
    m1 = lane >= 1
    ca0 = jnp.maximum(lane - 3, zero)
    cb0 = jnp.maximum(lane - 2, zero)
    cc0 = jnp.maximum(lane - 1, zero)
    ca1 = lane + 13
    cb1 = lane + 14
    cc1 = lane + 15
    node1 = lane + L
    n = ev_hbm.shape[1]

    for h in range(2):
        base = wid * rows + h * half
        pltpu.sync_copy(ev_hbm.at[pl.ds(base, half), :],
                        ev_v.at[:, pl.ds(0, n)])

        @plsc.parallel_loop(0, half, unroll=8)
        def row_body(r):
            rr = jnp.full((L,), r, jnp.int32)
            a = plsc.load_gather(ev_v, [rr, ca0])
            b = plsc.load_gather(ev_v, [rr, cb0])
            c = plsc.load_gather(ev_v, [rr, cc0])
            conf0 = (jnp.where(m3, a << 2, zero)
                     + jnp.where(m2, b << 1, zero)
                     + jnp.where(m1, c, zero))
            out_v[r, pl.ds(0, L)] = plsc.load_gather(tbl_v, [lane, conf0])
            a1 = plsc.load_gather(ev_v, [rr, ca1])
            b1 = plsc.load_gather(ev_v, [rr, cb1])
            c1 = plsc.load_gather(ev_v, [rr, cc1])
            conf1 = (a1 << 2) + (b1 << 1) + c1
            out_v[r, pl.ds(L, L)] = plsc.load_gather(tbl_v, [node1, conf1])

        pltpu.sync_copy(out_v.at[:, pl.ds(0, n)],
                        out_hbm.at[pl.ds(base, half), :])


def kernel(evidence_tensor, logits):
    B, n = evidence_tensor.shape
    ev = evidence_tensor.astype(jnp.int32)
    rows = B // NW

    mesh = plsc.VectorSubcoreMesh(core_axis_name="c", subcore_axis_name="s")
    return pl.kernel(
        _sc_body,
        out_type=jax.ShapeDtypeStruct((B, n), jnp.float32),
        mesh=mesh,
        compiler_params=pltpu.CompilerParams(
            needs_layout_passes=False,
            use_tc_tiling_on_sc=True,
        ),
        scratch_types=[
            pltpu.VMEM((rows // 2, n), jnp.int32),
            pltpu.VMEM((n, 8), jnp.float32),
            pltpu.VMEM((rows // 2, n), jnp.float32),
        ],
    )(ev, logits)


# speedup vs baseline: 1.6037x; 1.4165x over previous
import jax
import jax.numpy as jnp
from jax import lax
from jax.experimental import pallas as pl
from jax.experimental.pallas import tpu as pltpu
from jax.experimental.pallas import tpu_sc as plsc

N_NODES = 32
NC = 2
NS = 16
NW = NC * NS
L = 16


def _sc_body(ev_hbm, logits_hbm, out_hbm, ev_v, tbl_v, out_v):
    wid = lax.axis_index("s") * NC + lax.axis_index("c")
    seg = ev_hbm.shape[1] // NW
    base = wid * seg

    pltpu.sync_copy(ev_hbm.at[:, pl.ds(base, seg)], ev_v)
    pltpu.sync_copy(logits_hbm, tbl_v)

    one = jnp.ones((L,), jnp.float32)
    zero = jnp.zeros((L,), jnp.int32)
    seven = jnp.full((L,), 7, jnp.int32)

    for r in range(8):
        for o in (0, L):
            x = tbl_v[r, pl.ds(o, L)]
            tbl_v[r, pl.ds(o, L)] = one / (one + jnp.exp(-x))

    nodes = [jnp.full((L,), i, jnp.int32) for i in range(N_NODES)]

    @plsc.parallel_loop(0, seg // L, unroll=2)
    def vec_body(vb):
        off = vb * L
        conf = zero
        for i in range(N_NODES):
            out_v[i, pl.ds(off, L)] = plsc.load_gather(tbl_v, [conf, nodes[i]])
            if i + 1 < N_NODES:
                e = ev_v[i, pl.ds(off, L)]
                conf = ((conf << 1) | e) & seven

    pltpu.sync_copy(out_v, out_hbm.at[:, pl.ds(base, seg)])


def kernel(evidence_tensor, logits):
    B, n = evidence_tensor.shape
    ev_t = evidence_tensor.astype(jnp.int32).T
    logits_t = logits.T
    seg = B // NW

    mesh = plsc.VectorSubcoreMesh(core_axis_name="c", subcore_axis_name="s")
    out_t = pl.kernel(
        _sc_body,
        out_type=jax.ShapeDtypeStruct((n, B), jnp.float32),
        mesh=mesh,
        compiler_params=pltpu.CompilerParams(
            needs_layout_passes=False,
            use_tc_tiling_on_sc=True,
        ),
        scratch_types=[
            pltpu.VMEM((N_NODES, seg), jnp.int32),
            pltpu.VMEM((8, N_NODES), jnp.float32),
            pltpu.VMEM((N_NODES, seg), jnp.float32),
        ],
    )(ev_t, logits_t)
    return out_t.T
